# baseline (speedup 1.0000x reference)
import functools

import jax
import jax.numpy as jnp
from jax import lax
from jax.experimental import pallas as pl
from jax.experimental.pallas import tpu as pltpu
from jax.experimental.pallas import tpu_sc as plsc


def _embed(idx_flat, token_table, pos_ext, *, n_rows, d, n_workers, chunk,
           seq_len, nc):
    per_w = n_rows // n_workers
    n_chunks = per_w // chunk
    mesh = plsc.VectorSubcoreMesh(core_axis_name="c", subcore_axis_name="s")

    @functools.partial(
        pl.kernel,
        out_type=jax.ShapeDtypeStruct((n_rows, d), jnp.float32),
        mesh=mesh,
        scratch_types=[
            pltpu.VMEM((chunk,), jnp.int32),
            pltpu.VMEM((chunk, d), jnp.float32),
            pltpu.VMEM(pos_ext.shape, jnp.float32),
            pltpu.SemaphoreType.DMA,
        ],
        compiler_params=pltpu.CompilerParams(use_tc_tiling_on_sc=False),
    )
    def run(idx_hbm, tab_hbm, pos_hbm, out_hbm, idx_v, rows_v, pos_v, sem):
        wid = lax.axis_index("s") * nc + lax.axis_index("c")
        base_w = wid * per_w
        pltpu.sync_copy(pos_hbm, pos_v)

        def chunk_body(ci, carry):
            f0 = base_w + ci * chunk
            m = lax.rem(f0, seq_len)
            pltpu.sync_copy(idx_hbm.at[pl.ds(f0, chunk)], idx_v)
            pltpu.async_copy(tab_hbm.at[idx_v], rows_v, sem).wait()

            def add_body(r, c2):
                pr = m + r
                for cc in range(d // 16):
                    plsc.addupdate(
                        rows_v.at[r, pl.ds(cc * 16, 16)],
                        pos_v[pr, pl.ds(cc * 16, 16)],
                    )
                return c2

            lax.fori_loop(0, chunk, add_body, 0, unroll=8)
            pltpu.sync_copy(rows_v, out_hbm.at[pl.ds(f0, chunk)])
            return carry

        lax.fori_loop(0, n_chunks, chunk_body, 0)

    return run(idx_flat, token_table, pos_ext)


def kernel(input_ids, token_table, position_table):
    b, s = input_ids.shape
    v, d = token_table.shape
    n_rows = b * s
    chunk = 512
    info = plsc.get_sparse_core_info()
    nc, ns = info.num_cores, info.num_subcores
    n_workers = nc * ns

    reps = -(-(s + chunk) // s)
    pos_ext = jnp.concatenate([position_table[:s]] * reps, axis=0)[: s + chunk]

    idx_flat = input_ids.reshape(-1)
    out = _embed(idx_flat, token_table, pos_ext, n_rows=n_rows, d=d,
                 n_workers=n_workers, chunk=chunk, seq_len=s, nc=nc)
    return out.reshape(b, s, d)


# speedup vs baseline: 1.0860x; 1.0860x over previous
import functools

import jax
import jax.numpy as jnp
from jax import lax
from jax.experimental import pallas as pl
from jax.experimental.pallas import tpu as pltpu
from jax.experimental.pallas import tpu_sc as plsc


def _embed(idx_flat, token_table, pos_ext, *, n_rows, d, n_workers, chunk,
           seq_len, nc):
    per_w = n_rows // n_workers
    n_chunks = per_w // chunk
    assert n_chunks % 2 == 0
    mesh = plsc.VectorSubcoreMesh(core_axis_name="c", subcore_axis_name="s")

    @functools.partial(
        pl.kernel,
        out_type=jax.ShapeDtypeStruct((n_rows, d), jnp.float32),
        mesh=mesh,
        scratch_types=[
            pltpu.VMEM((chunk,), jnp.int32),
            pltpu.VMEM((chunk,), jnp.int32),
            pltpu.VMEM((chunk, d), jnp.float32),
            pltpu.VMEM((chunk, d), jnp.float32),
            pltpu.VMEM(pos_ext.shape, jnp.float32),
            pltpu.SemaphoreType.DMA,
            pltpu.SemaphoreType.DMA,
            pltpu.SemaphoreType.DMA,
            pltpu.SemaphoreType.DMA,
            pltpu.SemaphoreType.DMA,
            pltpu.SemaphoreType.DMA,
        ],
        compiler_params=pltpu.CompilerParams(use_tc_tiling_on_sc=False),
    )
    def run(idx_hbm, tab_hbm, pos_hbm, out_hbm, idx_a, idx_b, rows_a, rows_b, pos_v,
            sg_a, sg_b, si_a, si_b, so_a, so_b):
        wid = lax.axis_index("s") * nc + lax.axis_index("c")
        base_w = wid * per_w
        pltpu.sync_copy(pos_hbm, pos_v)
        pltpu.sync_copy(idx_hbm.at[pl.ds(base_w, chunk)], idx_a)
        pltpu.async_copy(tab_hbm.at[idx_a], rows_a, sg_a)
        pltpu.async_copy(idx_hbm.at[pl.ds(base_w + chunk, chunk)],
                         idx_b, si_b)

        def half(i, cur, idx_c, idx_n, rows_c, rows_n, sg_c, sg_n, si_c, si_n,
                 so_n):
            pltpu.make_async_copy(tab_hbm.at[idx_c], rows_c, sg_c).wait()

            @pl.when(i + 1 < n_chunks)
            def _():
                @pl.when(i >= 1)
                def _():
                    pltpu.make_async_copy(
                        rows_n, out_hbm.at[pl.ds(0, chunk)], so_n).wait()

                pltpu.make_async_copy(idx_hbm.at[pl.ds(0, chunk)],
                                      idx_n, si_n).wait()
                pltpu.async_copy(tab_hbm.at[idx_n], rows_n, sg_n)

            @pl.when(i + 2 < n_chunks)
            def _():
                pltpu.async_copy(
                    idx_hbm.at[pl.ds(base_w + (i + 2) * chunk, chunk)],
                    idx_c, si_c)

            f0 = base_w + i * chunk
            m = lax.rem(f0, seq_len)

            def add_body(r, c2):
                pr = m + r
                for cc in range(d // 16):
                    plsc.addupdate(
                        rows_c.at[r, pl.ds(cc * 16, 16)],
                        pos_v[pr, pl.ds(cc * 16, 16)],
                    )
                return c2

            lax.fori_loop(0, chunk, add_body, 0, unroll=8)
            pltpu.async_copy(rows_c, out_hbm.at[pl.ds(f0, chunk)],
                             so_a if cur == 0 else so_b)

        def pair_body(j, carry):
            half(2 * j, 0, idx_a, idx_b, rows_a, rows_b, sg_a, sg_b, si_a,
                 si_b, so_b)
            half(2 * j + 1, 1, idx_b, idx_a, rows_b, rows_a, sg_b, sg_a,
                 si_b, si_a, so_a)
            return carry

        lax.fori_loop(0, n_chunks // 2, pair_body, 0)
        pltpu.make_async_copy(rows_a, out_hbm.at[pl.ds(0, chunk)],
                              so_a).wait()
        pltpu.make_async_copy(rows_b, out_hbm.at[pl.ds(0, chunk)],
                              so_b).wait()

    return run(idx_flat, token_table, pos_ext)


def kernel(input_ids, token_table, position_table):
    b, s = input_ids.shape
    v, d = token_table.shape
    n_rows = b * s
    chunk = 512
    info = plsc.get_sparse_core_info()
    nc, ns = info.num_cores, info.num_subcores
    n_workers = nc * ns

    reps = -(-(s + chunk) // s)
    pos_ext = jnp.concatenate([position_table[:s]] * reps, axis=0)[: s + chunk]

    idx_flat = input_ids.reshape(-1)
    out = _embed(idx_flat, token_table, pos_ext, n_rows=n_rows, d=d,
                 n_workers=n_workers, chunk=chunk, seq_len=s, nc=nc)
    return out.reshape(b, s, d)
